# Initial kernel scaffold; baseline (speedup 1.0000x reference)
import jax
import jax.numpy as jnp
from jax.experimental import pallas as pl


def kernel(input, table):
    raise NotImplementedError("write your pallas kernel here")


# speedup vs baseline: 1.0156x; 1.0156x over previous
import functools
import math

import jax
import jax.numpy as jnp
from jax import lax
from jax.experimental import pallas as pl
from jax.experimental.pallas import tpu as pltpu
from jax.experimental.pallas import tpu_sc as plsc

EMBED_DIM = 32
SCALE = math.sqrt(float(EMBED_DIM))
NUM_CORES = 2
NUM_SUBCORES = 16
NUM_WORKERS = NUM_CORES * NUM_SUBCORES
LANES = 16


def _make_sc_lookup(batch: int, chunk: int):
    assert batch % NUM_WORKERS == 0
    b_per_w = batch // NUM_WORKERS
    assert b_per_w % chunk == 0
    n_chunks = b_per_w // chunk
    assert chunk % 8 == 0

    mesh = plsc.VectorSubcoreMesh(
        core_axis_name="c", subcore_axis_name="s",
        num_cores=NUM_CORES, num_subcores=NUM_SUBCORES)

    @functools.partial(
        pl.kernel,
        out_type=jax.ShapeDtypeStruct((batch, EMBED_DIM), jnp.float32),
        mesh=mesh,
        scratch_types=[
            pltpu.VMEM((chunk,), jnp.int32),
            pltpu.VMEM((chunk, EMBED_DIM), jnp.float32),
            pltpu.SemaphoreType.DMA,
        ],
        compiler_params=pltpu.CompilerParams(use_tc_tiling_on_sc=False),
    )
    def sc_lookup(table_hbm, idx_hbm, out_hbm, idx_v, rows_v, sem):
        wid = lax.axis_index("s") * NUM_CORES + lax.axis_index("c")
        base = wid * b_per_w

        def chunk_body(c, carry):
            off = base + c * chunk
            pltpu.sync_copy(idx_hbm.at[pl.ds(off, chunk)], idx_v)
            pltpu.async_copy(table_hbm.at[idx_v], rows_v, sem).wait()

            def scale_body(i, carry2):
                for h in range(EMBED_DIM // LANES):
                    sl = pl.ds(h * LANES, LANES)
                    rows_v[i, sl] = rows_v[i, sl] * SCALE
                return carry2

            lax.fori_loop(0, chunk, scale_body, 0, unroll=4)
            pltpu.sync_copy(rows_v, out_hbm.at[pl.ds(off, chunk)])
            return carry

        lax.fori_loop(0, n_chunks, chunk_body, 0)

    return sc_lookup


def kernel(input, table):
    idx = input.reshape(-1).astype(jnp.int32)
    batch = idx.shape[0]
    flat = _make_sc_lookup(batch, chunk=1024)(table, idx)
    return flat.reshape(*input.shape, EMBED_DIM)


# speedup vs baseline: 1.0485x; 1.0324x over previous
import functools
import math

import jax
import jax.numpy as jnp
from jax import lax
from jax.experimental import pallas as pl
from jax.experimental.pallas import tpu as pltpu
from jax.experimental.pallas import tpu_sc as plsc

EMBED_DIM = 32
SCALE = math.sqrt(float(EMBED_DIM))
NUM_CORES = 2
NUM_SUBCORES = 16
NUM_WORKERS = NUM_CORES * NUM_SUBCORES
LANES = 16


def _make_sc_lookup(batch: int, chunk: int):
    assert batch % NUM_WORKERS == 0
    b_per_w = batch // NUM_WORKERS
    assert b_per_w % chunk == 0
    n_chunks = b_per_w // chunk
    assert n_chunks % 2 == 0
    assert chunk % 8 == 0

    mesh = plsc.VectorSubcoreMesh(
        core_axis_name="c", subcore_axis_name="s",
        num_cores=NUM_CORES, num_subcores=NUM_SUBCORES)

    @functools.partial(
        pl.kernel,
        out_type=jax.ShapeDtypeStruct((batch, EMBED_DIM), jnp.float32),
        mesh=mesh,
        scratch_types=[
            pltpu.VMEM((n_chunks, chunk), jnp.int32),
            pltpu.VMEM((chunk, EMBED_DIM), jnp.float32),
            pltpu.VMEM((chunk, EMBED_DIM), jnp.float32),
            pltpu.SemaphoreType.DMA,
            pltpu.SemaphoreType.DMA,
        ],
        compiler_params=pltpu.CompilerParams(use_tc_tiling_on_sc=False),
    )
    def sc_lookup(table_hbm, idx_hbm, out_hbm, idx_v, rows0, rows1, sem0,
                  sem1):
        wid = lax.axis_index("s") * NUM_CORES + lax.axis_index("c")

        pltpu.sync_copy(idx_hbm.at[pl.ds(wid * n_chunks, n_chunks)], idx_v)

        def start_gather(g, rows, sem):
            pltpu.async_copy(table_hbm.at[idx_v.at[g]], rows, sem)

        def finish(g, rows, sem):
            pltpu.make_async_copy(table_hbm.at[idx_v.at[g]], rows, sem).wait()

            def scale_body(i, carry):
                for h in range(EMBED_DIM // LANES):
                    sl = pl.ds(h * LANES, LANES)
                    rows[i, sl] = rows[i, sl] * SCALE
                return carry

            lax.fori_loop(0, chunk, scale_body, 0, unroll=8)
            off = (wid * n_chunks + g) * chunk
            pltpu.sync_copy(rows, out_hbm.at[pl.ds(off, chunk)])

        start_gather(0, rows0, sem0)

        def pair_body(p, carry):
            g0 = 2 * p
            start_gather(g0 + 1, rows1, sem1)
            finish(g0, rows0, sem0)

            @pl.when(g0 + 2 < n_chunks)
            def _():
                start_gather(g0 + 2, rows0, sem0)

            finish(g0 + 1, rows1, sem1)
            return carry

        lax.fori_loop(0, n_chunks // 2, pair_body, 0)

    return sc_lookup


def kernel(input, table):
    chunk = 1280
    idx = input.reshape(-1, chunk).astype(jnp.int32)
    batch = idx.size
    flat = _make_sc_lookup(batch, chunk=chunk)(table, idx)
    return flat.reshape(*input.shape, EMBED_DIM)


# speedup vs baseline: 1.4272x; 1.3612x over previous
import functools
import math

import jax
import jax.numpy as jnp
from jax import lax
from jax.experimental import pallas as pl
from jax.experimental.pallas import tpu as pltpu
from jax.experimental.pallas import tpu_sc as plsc

EMBED_DIM = 32
SCALE = math.sqrt(float(EMBED_DIM))
NUM_CORES = 2
NUM_SUBCORES = 16
NUM_WORKERS = NUM_CORES * NUM_SUBCORES
LANES = 16


def _make_sc_lookup(n_rows: int, n_cols: int, chunk_rows: int):
    assert n_rows % NUM_WORKERS == 0
    rows_per_w = n_rows // NUM_WORKERS
    assert rows_per_w % chunk_rows == 0
    n_chunks = rows_per_w // chunk_rows
    assert n_chunks % 2 == 0
    chunk = chunk_rows * n_cols

    mesh = plsc.VectorSubcoreMesh(
        core_axis_name="c", subcore_axis_name="s",
        num_cores=NUM_CORES, num_subcores=NUM_SUBCORES)

    @functools.partial(
        pl.kernel,
        out_type=jax.ShapeDtypeStruct((n_rows, n_cols, EMBED_DIM),
                                      jnp.float32),
        mesh=mesh,
        scratch_types=[
            pltpu.VMEM((n_chunks, chunk), jnp.int32),
            pltpu.VMEM((chunk, EMBED_DIM), jnp.float32),
            pltpu.VMEM((chunk, EMBED_DIM), jnp.float32),
            pltpu.VMEM((chunk_rows, n_cols, EMBED_DIM), jnp.float32),
            pltpu.VMEM((chunk_rows, n_cols, EMBED_DIM), jnp.float32),
            pltpu.SemaphoreType.DMA,
            pltpu.SemaphoreType.DMA,
        ],
        compiler_params=pltpu.CompilerParams(use_tc_tiling_on_sc=False),
    )
    def sc_lookup(table_hbm, idx_hbm, out_hbm, idx_v, flat0, flat1, struct0,
                  struct1, sem0, sem1):
        wid = lax.axis_index("s") * NUM_CORES + lax.axis_index("c")
        r0 = wid * rows_per_w

        pltpu.sync_copy(idx_hbm.at[pl.ds(wid * n_chunks, n_chunks)], idx_v)

        def start_gather(g, flat, sem):
            pltpu.async_copy(table_hbm.at[idx_v.at[g]], flat, sem)

        def finish(g, flat, struct, sem):
            pltpu.make_async_copy(table_hbm.at[idx_v.at[g]], flat, sem).wait()

            def scale_row(r, carry):
                base = r * n_cols

                def scale_col(c, carry2):
                    for h in range(EMBED_DIM // LANES):
                        sl = pl.ds(h * LANES, LANES)
                        struct[r, c, sl] = flat[base + c, sl] * SCALE
                    return carry2

                lax.fori_loop(0, n_cols, scale_col, 0, unroll=5)
                return carry

            lax.fori_loop(0, chunk_rows, scale_row, 0)
            pltpu.sync_copy(struct,
                            out_hbm.at[pl.ds(r0 + g * chunk_rows, chunk_rows)])

        start_gather(0, flat0, sem0)

        def pair_body(p, carry):
            g0 = 2 * p
            start_gather(g0 + 1, flat1, sem1)
            finish(g0, flat0, struct0, sem0)

            @pl.when(g0 + 2 < n_chunks)
            def _():
                start_gather(g0 + 2, flat0, sem0)

            finish(g0 + 1, flat1, struct1, sem1)
            return carry

        lax.fori_loop(0, n_chunks // 2, pair_body, 0)

    return sc_lookup


def kernel(input, table):
    n_rows, n_cols = input.shape
    chunk_rows = 8
    idx = input.reshape(-1, chunk_rows * n_cols).astype(jnp.int32)
    return _make_sc_lookup(n_rows, n_cols, chunk_rows)(table, idx)
